# baseline (speedup 1.0000x reference)
import math

import jax
import jax.numpy as jnp
from jax import lax
from jax.experimental import pallas as pl
from jax.experimental.pallas import tpu as pltpu


_SCALE = math.sqrt(4096.0)
_R = 128
_NBUF = 8


def _ring_kernel(x_hbm, pos_hbm, o_hbm, xb, posb, sem_l, sem_s, sem_p):
    b, s, d = x_hbm.shape
    ns = s // _R
    n = ns * b

    def x_copy(i, slot):
        sb = i // b
        bb = lax.rem(i, b)
        return pltpu.make_async_copy(
            x_hbm.at[bb, pl.ds(sb * _R, _R), :], xb.at[slot], sem_l.at[slot]
        )

    def store_copy(i, slot):
        sb = i // b
        bb = lax.rem(i, b)
        return pltpu.make_async_copy(
            xb.at[slot], o_hbm.at[bb, pl.ds(sb * _R, _R), :], sem_s.at[slot]
        )

    def pos_copy(sb):
        par = lax.rem(sb, 2)
        return pltpu.make_async_copy(
            pos_hbm.at[pl.ds(sb * _R, _R), :], posb.at[par], sem_p.at[par]
        )

    pos_copy(0).start()
    for i in range(_NBUF - 1):
        x_copy(i, i).start()

    def body(i, _):
        slot = lax.rem(i, _NBUF)
        sb = i // b
        bb = lax.rem(i, b)

        @pl.when(bb == 0)
        def _():
            pos_copy(sb).wait()

            @pl.when(sb + 1 < ns)
            def _():
                pos_copy(sb + 1).start()

        x_copy(i, slot).wait()
        xv = xb[slot]
        pv = posb[lax.rem(sb, 2)]
        xb[slot] = xv * _SCALE + pv
        store_copy(i, slot).start()

        j = i + _NBUF - 1

        @pl.when(j < n)
        def _():
            jslot = lax.rem(j, _NBUF)

            @pl.when(i >= 1)
            def _():
                store_copy(j - _NBUF, jslot).wait()

            x_copy(j, jslot).start()

        return 0

    lax.fori_loop(0, n, body, 0)

    def drain(i, _):
        slot = lax.rem(i, _NBUF)
        store_copy(i, slot).wait()
        return 0

    lax.fori_loop(n - _NBUF, n, drain, 0)


@jax.jit
def kernel(inputs, pos_table):
    b, s, d = inputs.shape
    return pl.pallas_call(
        _ring_kernel,
        in_specs=[
            pl.BlockSpec(memory_space=pl.ANY),
            pl.BlockSpec(memory_space=pl.ANY),
        ],
        out_specs=pl.BlockSpec(memory_space=pl.ANY),
        out_shape=jax.ShapeDtypeStruct((b, s, d), inputs.dtype),
        scratch_shapes=[
            pltpu.VMEM((_NBUF, _R, d), jnp.float32),
            pltpu.VMEM((2, _R, d), jnp.float32),
            pltpu.SemaphoreType.DMA((_NBUF,)),
            pltpu.SemaphoreType.DMA((_NBUF,)),
            pltpu.SemaphoreType.DMA((2,)),
        ],
    )(inputs, pos_table)


# speedup vs baseline: 1.0171x; 1.0171x over previous
import math

import jax
import jax.numpy as jnp
from jax import lax
from jax.experimental import pallas as pl
from jax.experimental.pallas import tpu as pltpu


_SCALE = math.sqrt(4096.0)
_R = 512
_NBUF = 4


def _ring_kernel(x_hbm, pos_hbm, o_hbm, xb, posb, sem_l, sem_s, sem_p):
    b, s, d = x_hbm.shape
    ns = s // _R
    n = ns * b

    def x_copy(i, slot):
        sb = i // b
        bb = lax.rem(i, b)
        return pltpu.make_async_copy(
            x_hbm.at[bb, pl.ds(sb * _R, _R), :], xb.at[slot], sem_l.at[slot]
        )

    def store_copy(i, slot):
        sb = i // b
        bb = lax.rem(i, b)
        return pltpu.make_async_copy(
            xb.at[slot], o_hbm.at[bb, pl.ds(sb * _R, _R), :], sem_s.at[slot]
        )

    def pos_copy(sb):
        par = lax.rem(sb, 2)
        return pltpu.make_async_copy(
            pos_hbm.at[pl.ds(sb * _R, _R), :], posb.at[par], sem_p.at[par]
        )

    pos_copy(0).start()
    for i in range(_NBUF - 1):
        x_copy(i, i).start()

    def body(i, _):
        slot = lax.rem(i, _NBUF)
        sb = i // b
        bb = lax.rem(i, b)

        @pl.when(bb == 0)
        def _():
            pos_copy(sb).wait()

            @pl.when(sb + 1 < ns)
            def _():
                pos_copy(sb + 1).start()

        x_copy(i, slot).wait()
        xv = xb[slot]
        pv = posb[lax.rem(sb, 2)]
        xb[slot] = xv * _SCALE + pv
        store_copy(i, slot).start()

        j = i + _NBUF - 1

        @pl.when(j < n)
        def _():
            jslot = lax.rem(j, _NBUF)

            @pl.when(i >= 1)
            def _():
                store_copy(j - _NBUF, jslot).wait()

            x_copy(j, jslot).start()

        return 0

    lax.fori_loop(0, n, body, 0)

    def drain(i, _):
        slot = lax.rem(i, _NBUF)
        store_copy(i, slot).wait()
        return 0

    lax.fori_loop(n - _NBUF, n, drain, 0)


@jax.jit
def kernel(inputs, pos_table):
    b, s, d = inputs.shape
    return pl.pallas_call(
        _ring_kernel,
        in_specs=[
            pl.BlockSpec(memory_space=pl.ANY),
            pl.BlockSpec(memory_space=pl.ANY),
        ],
        out_specs=pl.BlockSpec(memory_space=pl.ANY),
        out_shape=jax.ShapeDtypeStruct((b, s, d), inputs.dtype),
        scratch_shapes=[
            pltpu.VMEM((_NBUF, _R, d), jnp.float32),
            pltpu.VMEM((2, _R, d), jnp.float32),
            pltpu.SemaphoreType.DMA((_NBUF,)),
            pltpu.SemaphoreType.DMA((_NBUF,)),
            pltpu.SemaphoreType.DMA((2,)),
        ],
    )(inputs, pos_table)


# speedup vs baseline: 1.0173x; 1.0002x over previous
import math

import jax
import jax.numpy as jnp
from jax import lax
from jax.experimental import pallas as pl
from jax.experimental.pallas import tpu as pltpu


_SCALE = math.sqrt(4096.0)
_R = 512
_NBUF = 5


def _ring_kernel(x_hbm, pos_hbm, o_hbm, xb, posb, sem_l, sem_s, sem_p):
    b, s, d = x_hbm.shape
    ns = s // _R
    n = ns * b

    def x_copy(i, slot):
        sb = i // b
        bb = lax.rem(i, b)
        return pltpu.make_async_copy(
            x_hbm.at[bb, pl.ds(sb * _R, _R), :], xb.at[slot], sem_l.at[slot]
        )

    def store_copy(i, slot):
        sb = i // b
        bb = lax.rem(i, b)
        return pltpu.make_async_copy(
            xb.at[slot], o_hbm.at[bb, pl.ds(sb * _R, _R), :], sem_s.at[slot]
        )

    def pos_copy(sb):
        par = lax.rem(sb, 2)
        return pltpu.make_async_copy(
            pos_hbm.at[pl.ds(sb * _R, _R), :], posb.at[par], sem_p.at[par]
        )

    pos_copy(0).start()
    for i in range(_NBUF - 1):
        x_copy(i, i).start()

    def body(i, _):
        slot = lax.rem(i, _NBUF)
        sb = i // b
        bb = lax.rem(i, b)

        @pl.when(bb == 0)
        def _():
            pos_copy(sb).wait()

            @pl.when(sb + 1 < ns)
            def _():
                pos_copy(sb + 1).start()

        x_copy(i, slot).wait()
        xv = xb[slot]
        pv = posb[lax.rem(sb, 2)]
        xb[slot] = xv * _SCALE + pv
        store_copy(i, slot).start()

        j = i + _NBUF - 1

        @pl.when(j < n)
        def _():
            jslot = lax.rem(j, _NBUF)

            @pl.when(i >= 1)
            def _():
                store_copy(j - _NBUF, jslot).wait()

            x_copy(j, jslot).start()

        return 0

    lax.fori_loop(0, n, body, 0)

    def drain(i, _):
        slot = lax.rem(i, _NBUF)
        store_copy(i, slot).wait()
        return 0

    lax.fori_loop(n - _NBUF, n, drain, 0)


@jax.jit
def kernel(inputs, pos_table):
    b, s, d = inputs.shape
    return pl.pallas_call(
        _ring_kernel,
        in_specs=[
            pl.BlockSpec(memory_space=pl.ANY),
            pl.BlockSpec(memory_space=pl.ANY),
        ],
        out_specs=pl.BlockSpec(memory_space=pl.ANY),
        out_shape=jax.ShapeDtypeStruct((b, s, d), inputs.dtype),
        scratch_shapes=[
            pltpu.VMEM((_NBUF, _R, d), jnp.float32),
            pltpu.VMEM((2, _R, d), jnp.float32),
            pltpu.SemaphoreType.DMA((_NBUF,)),
            pltpu.SemaphoreType.DMA((_NBUF,)),
            pltpu.SemaphoreType.DMA((2,)),
        ],
    )(inputs, pos_table)
